# baseline (speedup 1.0000x reference)
import jax
import jax.numpy as jnp
from jax import lax
from jax.experimental import pallas as pl
from jax.experimental.pallas import tpu as pltpu


def _build_M(nb, n):
    K = nb.shape[1]
    iot = lax.broadcasted_iota(jnp.int32, (n, n), 1)
    acc = jnp.zeros((n, n), jnp.float32)
    for k in range(K):
        acc = acc + (nb[:, k : k + 1] == iot).astype(jnp.float32)
    norm = jnp.sum((nb > -1).astype(jnp.float32), axis=1, keepdims=True)
    norm = jnp.where(norm == 0.0, 1.0, norm)
    return acc, norm


def _dot(a, b):
    return jax.lax.dot_general(
        a, b, (((1,), (0,)), ((), ())),
        preferred_element_type=jnp.float32,
    )


def _dot_hi(a, b):
    return jax.lax.dot_general(
        a, b, (((1,), (0,)), ((), ())),
        preferred_element_type=jnp.float32,
        precision=jax.lax.Precision.HIGHEST,
    )


def _dot_bf16(a, b):
    return jax.lax.dot_general(
        a.astype(jnp.bfloat16), b.astype(jnp.bfloat16),
        (((1,), (0,)), ((), ())),
        preferred_element_type=jnp.float32,
    )


def _body(z1_ref, z2_ref, nb1_ref, nb2_ref, wr0_ref, wnr0_ref,
          wr1_ref, wnr1_ref, wfc0_ref, bfc0_ref, wfc1_ref, bfc1_ref,
          out_ref, a_s, bt_s):
    i = pl.program_id(0)
    bi, n2 = out_ref.shape
    f1 = a_s.shape[1]

    @pl.when(i == 0)
    def _gnn():
        n = z1_ref.shape[0]
        m1, norm1 = _build_M(nb1_ref[...], n)
        m2, norm2 = _build_M(nb2_ref[...], n)
        wr0 = wr0_ref[...]
        wnr0 = wnr0_ref[...]
        wr1 = wr1_ref[...]
        wnr1 = wnr1_ref[...]

        def layer(x, m, norm, wr, wnr):
            nbs = _dot(x, wnr)
            return jax.nn.relu(_dot(x, wr) + _dot_hi(m, nbs) / norm)

        x1 = layer(z1_ref[...], m1, norm1, wr0, wnr0)
        x1 = layer(x1, m1, norm1, wr1, wnr1)
        x2 = layer(z2_ref[...], m2, norm2, wr0, wnr0)
        x2 = layer(x2, m2, norm2, wr1, wnr1)

        a_s[...] = _dot_bf16(x1, wfc0_ref[:f1, :])
        bt_s[...] = jnp.transpose(
            _dot_bf16(x2, wfc0_ref[f1:, :]) + bfc0_ref[...][None, :])

    a = a_s[pl.ds(i * bi, bi), :]
    acc = jnp.full((bi, n2), bfc1_ref[0], jnp.float32)
    for c in range(f1):
        wc = lax.convert_element_type(
            lax.convert_element_type(wfc1_ref[c, 0], jnp.bfloat16), jnp.float32)
        t = jnp.maximum(a[:, c : c + 1] + bt_s[c : c + 1, :], 0.0)
        acc = acc + t * wc
    out_ref[...] = acc


def kernel(Z1, Z2, neighbors1, neighbors2, Wr0, Wnr0, Wr1, Wnr1,
           W_fc0, b_fc0, W_fc1, b_fc1):
    n1, _ = Z1.shape
    n2, _ = Z2.shape
    f1 = W_fc1.shape[0]

    bi = 32
    full = lambda shape: pl.BlockSpec(shape, lambda i: tuple(0 for _ in shape))
    out2d = pl.pallas_call(
        _body,
        grid=(n1 // bi,),
        in_specs=[
            pl.BlockSpec(memory_space=pltpu.VMEM),
            pl.BlockSpec(memory_space=pltpu.VMEM),
            pl.BlockSpec(memory_space=pltpu.VMEM),
            pl.BlockSpec(memory_space=pltpu.VMEM),
            pl.BlockSpec(memory_space=pltpu.VMEM),
            pl.BlockSpec(memory_space=pltpu.VMEM),
            pl.BlockSpec(memory_space=pltpu.VMEM),
            pl.BlockSpec(memory_space=pltpu.VMEM),
            pl.BlockSpec(memory_space=pltpu.VMEM),
            pl.BlockSpec(memory_space=pltpu.VMEM),
            pl.BlockSpec(memory_space=pltpu.SMEM),
            pl.BlockSpec(memory_space=pltpu.SMEM),
        ],
        out_specs=pl.BlockSpec((bi, n2), lambda i: (i, 0)),
        out_shape=jax.ShapeDtypeStruct((n1, n2), jnp.float32),
        scratch_shapes=[
            pltpu.VMEM((n1, f1), jnp.float32),
            pltpu.VMEM((f1, n2), jnp.float32),
        ],
    )(Z1, Z2, neighbors1, neighbors2, Wr0, Wnr0, Wr1, Wnr1,
      W_fc0, b_fc0, W_fc1, b_fc1)

    return out2d.reshape(n1 * n2)
